# baseline (speedup 1.0000x reference)
import functools

import jax
import jax.numpy as jnp
from jax import lax
from jax.experimental import pallas as pl
from jax.experimental.pallas import tpu as pltpu
from jax.experimental.pallas import tpu_sc as plsc

TILE_N = 128
SC_NUM_CORES = 2
SC_NUM_SUBCORES = 16


def _vq_tile_kernel(x_ref, embed_ref, embt_ref, dist_ref, onehot_ref,
                    ind_ref):
    emb = embed_ref[...]
    embt = embt_ref[...]
    e2 = jnp.sum(embt * embt, axis=0, keepdims=True)
    x = x_ref[...]
    x2 = jnp.sum(x * x, axis=-1, keepdims=True)
    xe = jax.lax.dot_general(
        x, emb, (((1,), (1,)), ((), ())),
        preferred_element_type=jnp.float32)
    dist = -(x2 - 2.0 * xe + e2)
    dist_ref[...] = dist
    ind = jnp.argmax(dist, axis=-1)
    iota = jax.lax.broadcasted_iota(jnp.int32, dist.shape, 1)
    onehot_ref[...] = (iota == ind[:, None]).astype(jnp.float32)
    ind_ref[0, 0, :] = ind


def _make_sc_gather(n, d):
    nw = SC_NUM_CORES * SC_NUM_SUBCORES
    b_per_w = n // nw
    mesh = plsc.VectorSubcoreMesh(
        core_axis_name="c", subcore_axis_name="s",
        num_cores=SC_NUM_CORES, num_subcores=SC_NUM_SUBCORES)

    @functools.partial(
        pl.kernel, mesh=mesh,
        out_type=jax.ShapeDtypeStruct((n, d), jnp.float32),
        scratch_types=[
            pltpu.VMEM((b_per_w,), jnp.int32),
            pltpu.VMEM((b_per_w, d), jnp.float32),
            pltpu.SemaphoreType.DMA,
        ],
        compiler_params=pltpu.CompilerParams(use_tc_tiling_on_sc=False),
    )
    def gather_rows(table_hbm, idx_hbm, out_hbm, idx_v, rows_v, sem):
        wid = lax.axis_index("s") * SC_NUM_CORES + lax.axis_index("c")
        base = wid * b_per_w
        pltpu.sync_copy(idx_hbm.at[pl.ds(base, b_per_w)], idx_v)
        pltpu.async_copy(table_hbm.at[idx_v], rows_v, sem).wait()
        pltpu.sync_copy(rows_v, out_hbm.at[pl.ds(base, b_per_w)])

    return gather_rows


def kernel(x, embed):
    x = x.astype(jnp.float32)
    b, t, d = x.shape
    n = b * t
    c = embed.shape[1]
    n_tiles = n // TILE_N
    xf = x.reshape(n, d)
    emb = embed.reshape(c, d)
    embt = emb.T

    dist, onehot, ind = pl.pallas_call(
        _vq_tile_kernel,
        grid=(n_tiles,),
        in_specs=[
            pl.BlockSpec((TILE_N, d), lambda i: (i, 0)),
            pl.BlockSpec((c, d), lambda i: (0, 0)),
            pl.BlockSpec((d, c), lambda i: (0, 0)),
        ],
        out_specs=[
            pl.BlockSpec((TILE_N, c), lambda i: (i, 0)),
            pl.BlockSpec((TILE_N, c), lambda i: (i, 0)),
            pl.BlockSpec((1, 1, TILE_N), lambda i: (i, 0, 0)),
        ],
        out_shape=[
            jax.ShapeDtypeStruct((n, c), jnp.float32),
            jax.ShapeDtypeStruct((n, c), jnp.float32),
            jax.ShapeDtypeStruct((n_tiles, 1, TILE_N), jnp.int32),
        ],
        compiler_params=pltpu.CompilerParams(
            dimension_semantics=("parallel",)),
    )(xf, emb, embt)

    ind_flat = ind.reshape(n)
    quant = _make_sc_gather(n, d)(emb, ind_flat)

    embed_ind = ind_flat.reshape(b, t)
    quantize = quant.reshape(b, t, d)
    embed_onehot = onehot.reshape(1, n, c)
    dist_out = dist.reshape(1, b, t, c)
    return (quantize, embed_ind, embed_onehot, dist_out)
